# Initial kernel scaffold; baseline (speedup 1.0000x reference)
import functools

import jax
import jax.numpy as jnp
from jax import lax
from jax.experimental import pallas as pl
from jax.experimental.pallas import tpu as pltpu
from jax.experimental.pallas import tpu_sc as plsc

N = 10000
E = 320000
D = 128
NC, NS = 2, 16
NW = NC * NS
EPW = E // NW
CH = 128
NFULL = EPW // CH
TAIL = EPW - NFULL * CH
RPW = N // NS
ZR = 125
NBLK = 2000
EBLK = 4000

f32 = jnp.float32
_mesh = plsc.VectorSubcoreMesh(core_axis_name="c", subcore_axis_name="s")


@functools.partial(
    pl.kernel,
    out_type=(jax.ShapeDtypeStruct((E, D), f32),
              jax.ShapeDtypeStruct((E, D), f32)),
    mesh=_mesh,
    scratch_types=[
        pltpu.VMEM((CH,), jnp.int32), pltpu.VMEM((CH,), jnp.int32),
        pltpu.VMEM((CH, D), f32), pltpu.VMEM((CH, D), f32),
        pltpu.VMEM((TAIL,), jnp.int32), pltpu.VMEM((TAIL,), jnp.int32),
        pltpu.VMEM((TAIL, D), f32), pltpu.VMEM((TAIL, D), f32),
        pltpu.SemaphoreType.DMA,
    ],
)
def _gather_sc(pi_hbm, pj_hbm, dst_hbm, src_hbm, gi_hbm, gj_hbm,
               ii, ij, ri, rj, tii, tij, tri, trj, sem):
    wid = lax.axis_index("s") * NC + lax.axis_index("c")
    e0 = wid * EPW

    def chunk(base, iiv, ijv, riv, rjv, n):
        pltpu.sync_copy(dst_hbm.at[pl.ds(base, n)], iiv)
        pltpu.sync_copy(src_hbm.at[pl.ds(base, n)], ijv)
        c1 = pltpu.async_copy(pi_hbm.at[iiv], riv, sem)
        c2 = pltpu.async_copy(pj_hbm.at[ijv], rjv, sem)
        c1.wait()
        c2.wait()
        pltpu.sync_copy(riv, gi_hbm.at[pl.ds(base, n)])
        pltpu.sync_copy(rjv, gj_hbm.at[pl.ds(base, n)])

    def body(k, carry):
        chunk(pl.multiple_of(e0 + k * CH, CH), ii, ij, ri, rj, CH)
        return carry

    lax.fori_loop(0, NFULL, body, 0)
    chunk(e0 + NFULL * CH, tii, tij, tri, trj, TAIL)


@functools.partial(
    pl.kernel,
    out_type=jax.ShapeDtypeStruct((NC * N, D), f32),
    mesh=_mesh,
    scratch_types=[
        pltpu.VMEM((CH,), jnp.int32), pltpu.VMEM((CH, D), f32),
        pltpu.VMEM((TAIL,), jnp.int32), pltpu.VMEM((TAIL, D), f32),
        pltpu.VMEM((ZR, D), f32),
        pltpu.VMEM_SHARED((N, D), f32),
        pltpu.SemaphoreType.DMA,
    ],
)
def _scatter_sc(m_hbm, dst_hbm, zeros_hbm, out_hbm,
                idx, rows, tidx, trows, stage, acc, sem):
    cid = lax.axis_index("c")
    sid = lax.axis_index("s")
    wid = sid * NC + cid
    r0 = sid * RPW

    pltpu.sync_copy(zeros_hbm, stage)
    for t in range(RPW // ZR):
        pltpu.sync_copy(stage, acc.at[pl.ds(r0 + t * ZR, ZR)])
    plsc.subcore_barrier()

    e0 = wid * EPW

    def chunk(base, iv, rv, n):
        pltpu.sync_copy(dst_hbm.at[pl.ds(base, n)], iv)
        pltpu.sync_copy(m_hbm.at[pl.ds(base, n)], rv)
        pltpu.sync_copy(rv, acc.at[iv], add=True)

    def body(k, carry):
        chunk(pl.multiple_of(e0 + k * CH, CH), idx, rows, CH)
        return carry

    lax.fori_loop(0, NFULL, body, 0)
    chunk(e0 + NFULL * CH, tidx, trows, TAIL)
    plsc.subcore_barrier()

    for t in range(RPW // ZR):
        pltpu.sync_copy(acc.at[pl.ds(r0 + t * ZR, ZR)], stage)
        pltpu.sync_copy(stage, out_hbm.at[pl.ds(cid * N + r0 + t * ZR, ZR)])


def _dot(a, b):
    return jnp.dot(a, b, preferred_element_type=f32)


def _ln(h, g, b):
    mu = jnp.mean(h, axis=1, keepdims=True)
    d = h - mu
    var = jnp.mean(d * d, axis=1, keepdims=True)
    return d * lax.rsqrt(var + 1e-5) * g + b


def _mlp3(h, W1, b1, W2, b2, W3, b3):
    h = jax.nn.relu(_dot(h, W1[...]) + b1[...])
    h = jax.nn.relu(_dot(h, W2[...]) + b2[...])
    return _dot(h, W3[...]) + b3[...]


def _enc_node_body(x, W1, b1, W2, b2, W3, b3, g, be, Wi, Wj, xo, pio, pjo):
    xn = _ln(_mlp3(x[...], W1, b1, W2, b2, W3, b3), g[...], be[...])
    xo[...] = xn
    pio[...] = _dot(xn, Wi[...])
    pjo[...] = _dot(xn, Wj[...])


def _edge1_body(ea, gi, gj,
                eW1, eb1, eW2, eb2, eW3, eb3, eg, ebe,
                W1e, b1, W2, b2, W3, b3, g, be, mo):
    eh = _ln(_mlp3(ea[...], eW1, eb1, eW2, eb2, eW3, eb3), eg[...], ebe[...])
    h = jax.nn.relu(_dot(eh, W1e[...]) + gi[...] + gj[...] + b1[...])
    h = jax.nn.relu(_dot(h, W2[...]) + b2[...])
    h = _dot(h, W3[...]) + b3[...]
    mo[...] = _ln(h, g[...], be[...])


def _edge_body(eh, gi, gj, W1e, b1, W2, b2, W3, b3, g, be, mo):
    h = jax.nn.relu(_dot(eh[...], W1e[...]) + gi[...] + gj[...] + b1[...])
    h = jax.nn.relu(_dot(h, W2[...]) + b2[...])
    h = _dot(h, W3[...]) + b3[...]
    mo[...] = _ln(h, g[...], be[...])


def _node_upd_body(p0, p1, x, W1a, W1x, b1, W2, b2, W3, b3, g, be, Wi, Wj,
                   xo, pio, pjo):
    agg = p0[...] + p1[...]
    h = jax.nn.relu(_dot(agg, W1a[...]) + _dot(x[...], W1x[...]) + b1[...])
    h = jax.nn.relu(_dot(h, W2[...]) + b2[...])
    h = _dot(h, W3[...]) + b3[...]
    xn = _ln(h, g[...], be[...])
    xo[...] = xn
    pio[...] = _dot(xn, Wi[...])
    pjo[...] = _dot(xn, Wj[...])


def _node_dec_body(p0, p1, x, W1a, W1x, b1, W2, b2, W3, b3, g, be,
                   dW1, db1, dW2, db2, dW3, db3, oo):
    agg = p0[...] + p1[...]
    h = jax.nn.relu(_dot(agg, W1a[...]) + _dot(x[...], W1x[...]) + b1[...])
    h = jax.nn.relu(_dot(h, W2[...]) + b2[...])
    h = _dot(h, W3[...]) + b3[...]
    xn = _ln(h, g[...], be[...])
    oo[...] = _mlp3(xn, dW1, db1, dW2, db2, dW3, db3)


def _rows(blk, off_blocks=0):
    if off_blocks:
        return pl.BlockSpec((blk, D), lambda i: (off_blocks + i, 0))
    return pl.BlockSpec((blk, D), lambda i: (i, 0))


def _wspec(arr):
    nd = arr.ndim
    return pl.BlockSpec(arr.shape, lambda i: (0,) * nd)


def _tc_call(body, grid, row_ins, weights, n_out, out_rows, blk):
    return pl.pallas_call(
        body,
        grid=(grid,),
        in_specs=list(row_ins) + [_wspec(w) for w in weights],
        out_specs=[_rows(blk)] * n_out,
        out_shape=[jax.ShapeDtypeStruct((out_rows, D), f32)] * n_out,
    )


def kernel(x, edge_index, edge_attr, params):
    src = edge_index[0]
    dst = edge_index[1]

    def vw(v):
        return v.reshape(1, D)

    def unpack(mp, ln=True):
        (W1, b1), (W2, b2), (W3, b3) = mp["layers"]
        out = [W1, vw(b1), W2, vw(b2), W3, vw(b3)]
        if ln:
            g, be = mp["ln"]
            out += [vw(g), vw(be)]
        return out

    enc_n = unpack(params["node_enc"])
    enc_e = unpack(params["edge_enc"])
    upd_e = unpack(params["edge_upd"])
    upd_n = unpack(params["node_upd"])
    dec = unpack(params["node_dec"], ln=False)

    W1u = upd_e[0]
    W1e, W1i, W1j = W1u[:D], W1u[D:2 * D], W1u[2 * D:]
    upd_e_k = [W1e] + upd_e[1:]
    W1n = upd_n[0]
    upd_n_k = [W1n[:D], W1n[D:]] + upd_n[1:]
    zeros = jnp.zeros((ZR, D), f32)

    ng = N // NBLK
    eg = E // EBLK

    enc_node = _tc_call(_enc_node_body, ng, [_rows(NBLK)],
                        enc_n + [W1i, W1j], 3, N, NBLK)
    ea_spec = pl.BlockSpec((EBLK, edge_attr.shape[1]), lambda i: (i, 0))
    edge1 = _tc_call(_edge1_body, eg, [ea_spec, _rows(EBLK), _rows(EBLK)],
                     enc_e + upd_e_k, 1, E, EBLK)
    edge = _tc_call(_edge_body, eg, [_rows(EBLK)] * 3, upd_e_k, 1, E, EBLK)
    node_upd = _tc_call(_node_upd_body, ng,
                        [_rows(NBLK), _rows(NBLK, N // NBLK), _rows(NBLK)],
                        upd_n_k + [W1i, W1j], 3, N, NBLK)
    node_dec = _tc_call(_node_dec_body, ng,
                        [_rows(NBLK), _rows(NBLK, N // NBLK), _rows(NBLK)],
                        upd_n_k + dec, 1, N, NBLK)

    X, Pi, Pj = enc_node(x, *enc_n, W1i, W1j)
    m = None
    out = None
    for step in range(4):
        Gi, Gj = _gather_sc(Pi, Pj, dst, src)
        if step == 0:
            (m,) = edge1(edge_attr, Gi, Gj, *enc_e, *upd_e_k)
        else:
            (m,) = edge(m, Gi, Gj, *upd_e_k)
        parts = _scatter_sc(m, dst, zeros)
        if step < 3:
            X, Pi, Pj = node_upd(parts, parts, X, *upd_n_k, W1i, W1j)
        else:
            (out,) = node_dec(parts, parts, X, *upd_n_k, *dec)
    return out


# speedup vs baseline: 3.5979x; 3.5979x over previous
import functools

import jax
import jax.numpy as jnp
from jax import lax
from jax.experimental import pallas as pl
from jax.experimental.pallas import tpu as pltpu
from jax.experimental.pallas import tpu_sc as plsc

N = 10000
E = 320000
D = 128
NC, NS = 2, 16
NW = NC * NS
EPW = E // NW
CH = 128
NFULL = EPW // CH
TAIL = EPW - NFULL * CH
NP = 10240
RPW = NP // NS
ZR = 128
NBLK = 2000
EBLK = 4000

f32 = jnp.float32


@functools.cache
def _sc_kernels():
    mesh = plsc.VectorSubcoreMesh(core_axis_name="c", subcore_axis_name="s",
                                  num_cores=NC, num_subcores=NS)

    @functools.partial(
        pl.kernel,
        out_type=(jax.ShapeDtypeStruct((E, D), f32),
                  jax.ShapeDtypeStruct((E, D), f32)),
        mesh=mesh,
        scratch_types=[
            pltpu.VMEM((CH,), jnp.int32), pltpu.VMEM((CH,), jnp.int32),
            pltpu.VMEM((CH, D), f32), pltpu.VMEM((CH, D), f32),
            pltpu.VMEM((TAIL,), jnp.int32), pltpu.VMEM((TAIL,), jnp.int32),
            pltpu.VMEM((TAIL, D), f32), pltpu.VMEM((TAIL, D), f32),
            pltpu.SemaphoreType.DMA,
        ],
    )
    def _gather_sc(pi_hbm, pj_hbm, dst_hbm, src_hbm, gi_hbm, gj_hbm,
                   ii, ij, ri, rj, tii, tij, tri, trj, sem):
        wid = lax.axis_index("s") * NC + lax.axis_index("c")
        e0 = wid * EPW

        def chunk(base, iiv, ijv, riv, rjv, n):
            pltpu.sync_copy(dst_hbm.at[pl.ds(base, n)], iiv)
            pltpu.sync_copy(src_hbm.at[pl.ds(base, n)], ijv)
            c1 = pltpu.async_copy(pi_hbm.at[iiv], riv, sem)
            c2 = pltpu.async_copy(pj_hbm.at[ijv], rjv, sem)
            c1.wait()
            c2.wait()
            pltpu.sync_copy(riv, gi_hbm.at[pl.ds(base, n)])
            pltpu.sync_copy(rjv, gj_hbm.at[pl.ds(base, n)])

        def body(k, carry):
            chunk(pl.multiple_of(e0 + k * CH, 16), ii, ij, ri, rj, CH)
            return carry

        lax.fori_loop(0, NFULL, body, 0)
        chunk(pl.multiple_of(e0 + NFULL * CH, 16), tii, tij, tri, trj, TAIL)

    @functools.partial(
        pl.kernel,
        out_type=jax.ShapeDtypeStruct((2 * NP, D), f32),
        mesh=mesh,
        scratch_types=[
            pltpu.VMEM((CH,), jnp.int32), pltpu.VMEM((CH, D), f32),
            pltpu.VMEM((TAIL,), jnp.int32), pltpu.VMEM((TAIL, D), f32),
            pltpu.VMEM((ZR, D), f32),
            pltpu.VMEM_SHARED((NP, D), f32),
            pltpu.SemaphoreType.DMA,
        ],
    )
    def _scatter_sc(m_hbm, dst_hbm, zeros_hbm, out_hbm,
                    idx, rows, tidx, trows, stage, acc, sem):
        cid = lax.axis_index("c")
        sid = lax.axis_index("s")
        wid = sid * NC + cid
        r0 = sid * RPW

        pltpu.sync_copy(zeros_hbm, stage)
        for t in range(RPW // ZR):
            pltpu.sync_copy(stage, acc.at[pl.ds(r0 + t * ZR, ZR)])
        plsc.subcore_barrier()

        e0 = wid * EPW

        def chunk(base, iv, rv, n):
            pltpu.sync_copy(dst_hbm.at[pl.ds(base, n)], iv)
            pltpu.sync_copy(m_hbm.at[pl.ds(base, n)], rv)
            pltpu.sync_copy(rv, acc.at[iv], add=True)

        def body(k, carry):
            chunk(pl.multiple_of(e0 + k * CH, 16), idx, rows, CH)
            return carry

        lax.fori_loop(0, NFULL, body, 0)
        chunk(pl.multiple_of(e0 + NFULL * CH, 16), tidx, trows, TAIL)
        plsc.subcore_barrier()

        for t in range(RPW // ZR):
            pltpu.sync_copy(acc.at[pl.ds(r0 + t * ZR, ZR)], stage)
            pltpu.sync_copy(
                stage,
                out_hbm.at[pl.ds(pl.multiple_of(cid * NP + r0 + t * ZR, 8),
                                 ZR)])

    return _gather_sc, _scatter_sc


def _dot(a, b):
    return jnp.dot(a, b, preferred_element_type=f32)


def _ln(h, g, b):
    mu = jnp.mean(h, axis=1, keepdims=True)
    d = h - mu
    var = jnp.mean(d * d, axis=1, keepdims=True)
    return d * lax.rsqrt(var + 1e-5) * g + b


def _mlp3(h, W1, b1, W2, b2, W3, b3):
    h = jax.nn.relu(_dot(h, W1[...]) + b1[...])
    h = jax.nn.relu(_dot(h, W2[...]) + b2[...])
    return _dot(h, W3[...]) + b3[...]


def _enc_node_body(x, W1, b1, W2, b2, W3, b3, g, be, Wi, Wj, xo, pio, pjo):
    xn = _ln(_mlp3(x[...], W1, b1, W2, b2, W3, b3), g[...], be[...])
    xo[...] = xn
    pio[...] = _dot(xn, Wi[...])
    pjo[...] = _dot(xn, Wj[...])


def _edge1_body(ea, gi, gj,
                eW1, eb1, eW2, eb2, eW3, eb3, eg, ebe,
                W1e, b1, W2, b2, W3, b3, g, be, mo):
    eh = _ln(_mlp3(ea[...], eW1, eb1, eW2, eb2, eW3, eb3), eg[...], ebe[...])
    h = jax.nn.relu(_dot(eh, W1e[...]) + gi[...] + gj[...] + b1[...])
    h = jax.nn.relu(_dot(h, W2[...]) + b2[...])
    h = _dot(h, W3[...]) + b3[...]
    mo[...] = _ln(h, g[...], be[...])


def _edge_body(eh, gi, gj, W1e, b1, W2, b2, W3, b3, g, be, mo):
    h = jax.nn.relu(_dot(eh[...], W1e[...]) + gi[...] + gj[...] + b1[...])
    h = jax.nn.relu(_dot(h, W2[...]) + b2[...])
    h = _dot(h, W3[...]) + b3[...]
    mo[...] = _ln(h, g[...], be[...])


def _node_upd_body(p0, p1, x, W1a, W1x, b1, W2, b2, W3, b3, g, be, Wi, Wj,
                   xo, pio, pjo):
    agg = p0[...] + p1[...]
    h = jax.nn.relu(_dot(agg, W1a[...]) + _dot(x[...], W1x[...]) + b1[...])
    h = jax.nn.relu(_dot(h, W2[...]) + b2[...])
    h = _dot(h, W3[...]) + b3[...]
    xn = _ln(h, g[...], be[...])
    xo[...] = xn
    pio[...] = _dot(xn, Wi[...])
    pjo[...] = _dot(xn, Wj[...])


def _node_dec_body(p0, p1, x, W1a, W1x, b1, W2, b2, W3, b3, g, be,
                   dW1, db1, dW2, db2, dW3, db3, oo):
    agg = p0[...] + p1[...]
    h = jax.nn.relu(_dot(agg, W1a[...]) + _dot(x[...], W1x[...]) + b1[...])
    h = jax.nn.relu(_dot(h, W2[...]) + b2[...])
    h = _dot(h, W3[...]) + b3[...]
    xn = _ln(h, g[...], be[...])
    oo[...] = _mlp3(xn, dW1, db1, dW2, db2, dW3, db3)


def _rows(blk, off_blocks=0):
    if off_blocks:
        return pl.BlockSpec((blk, D), lambda i: (off_blocks + i, 0))
    return pl.BlockSpec((blk, D), lambda i: (i, 0))


def _wspec(arr):
    nd = arr.ndim
    return pl.BlockSpec(arr.shape, lambda i: (0,) * nd)


def _tc_call(body, grid, row_ins, weights, n_out, out_rows, blk):
    return pl.pallas_call(
        body,
        grid=(grid,),
        in_specs=list(row_ins) + [_wspec(w) for w in weights],
        out_specs=[_rows(blk)] * n_out,
        out_shape=[jax.ShapeDtypeStruct((out_rows, D), f32)] * n_out,
    )


def kernel(x, edge_index, edge_attr, params):
    src = edge_index[0]
    dst = edge_index[1]

    def vw(v):
        return v.reshape(1, D)

    def unpack(mp, ln=True):
        (W1, b1), (W2, b2), (W3, b3) = mp["layers"]
        out = [W1, vw(b1), W2, vw(b2), W3, vw(b3)]
        if ln:
            g, be = mp["ln"]
            out += [vw(g), vw(be)]
        return out

    enc_n = unpack(params["node_enc"])
    enc_e = unpack(params["edge_enc"])
    upd_e = unpack(params["edge_upd"])
    upd_n = unpack(params["node_upd"])
    dec = unpack(params["node_dec"], ln=False)

    W1u = upd_e[0]
    W1e, W1i, W1j = W1u[:D], W1u[D:2 * D], W1u[2 * D:]
    upd_e_k = [W1e] + upd_e[1:]
    W1n = upd_n[0]
    upd_n_k = [W1n[:D], W1n[D:]] + upd_n[1:]
    zeros = jnp.zeros((ZR, D), f32)

    ng = N // NBLK
    eg = E // EBLK

    enc_node = _tc_call(_enc_node_body, ng, [_rows(NBLK)],
                        enc_n + [W1i, W1j], 3, N, NBLK)
    ea_spec = pl.BlockSpec((EBLK, edge_attr.shape[1]), lambda i: (i, 0))
    edge1 = _tc_call(_edge1_body, eg, [ea_spec, _rows(EBLK), _rows(EBLK)],
                     enc_e + upd_e_k, 1, E, EBLK)
    edge = _tc_call(_edge_body, eg, [_rows(EBLK)] * 3, upd_e_k, 1, E, EBLK)
    node_upd = _tc_call(_node_upd_body, ng,
                        [_rows(NBLK), _rows(NBLK), _rows(NBLK)],
                        upd_n_k + [W1i, W1j], 3, N, NBLK)
    node_dec = _tc_call(_node_dec_body, ng,
                        [_rows(NBLK), _rows(NBLK), _rows(NBLK)],
                        upd_n_k + dec, 1, N, NBLK)

    gather_sc, scatter_sc = _sc_kernels()

    X, Pi, Pj = enc_node(x, *enc_n, W1i, W1j)
    m = None
    out = None
    for step in range(4):
        Gi, Gj = gather_sc(Pi, Pj, dst, src)
        if step == 0:
            (m,) = edge1(edge_attr, Gi, Gj, *enc_e, *upd_e_k)
        else:
            (m,) = edge(m, Gi, Gj, *upd_e_k)
        parts = scatter_sc(m, dst, zeros)
        p0 = lax.slice(parts, (0, 0), (N, D))
        p1 = lax.slice(parts, (NP, 0), (NP + N, D))
        if step < 3:
            X, Pi, Pj = node_upd(p0, p1, X, *upd_n_k, W1i, W1j)
        else:
            (out,) = node_dec(p0, p1, X, *upd_n_k, *dec)
    return out


# speedup vs baseline: 3.6879x; 1.0250x over previous
import functools

import jax
import jax.numpy as jnp
from jax import lax
from jax.experimental import pallas as pl
from jax.experimental.pallas import tpu as pltpu
from jax.experimental.pallas import tpu_sc as plsc

N = 10000
E = 320000
D = 128
NC, NS = 2, 16
NW = NC * NS
EPW = E // NW
CH = 128
NFULL = EPW // CH
TAIL = EPW - NFULL * CH
NP = 10240
RPW = NP // NS
ZR = 128
NBLK = 2000
EBLK = 4000

f32 = jnp.float32


@functools.cache
def _sc_kernels():
    mesh = plsc.VectorSubcoreMesh(core_axis_name="c", subcore_axis_name="s",
                                  num_cores=NC, num_subcores=NS)

    @functools.partial(
        pl.kernel,
        out_type=jax.ShapeDtypeStruct((E, D), f32),
        mesh=mesh,
        scratch_types=[
            pltpu.VMEM((CH,), jnp.int32), pltpu.VMEM((CH,), jnp.int32),
            pltpu.VMEM((CH, D), f32),
            pltpu.VMEM((TAIL,), jnp.int32), pltpu.VMEM((TAIL,), jnp.int32),
            pltpu.VMEM((TAIL, D), f32),
            pltpu.SemaphoreType.DMA,
        ],
    )
    def _gather_sc(pi_hbm, pj_hbm, dst_hbm, src_hbm, g_hbm,
                   ii, ij, ri, tii, tij, tri, sem):
        wid = lax.axis_index("s") * NC + lax.axis_index("c")
        e0 = wid * EPW

        def chunk(base, iiv, ijv, riv, n):
            pltpu.sync_copy(dst_hbm.at[pl.ds(base, n)], iiv)
            pltpu.sync_copy(src_hbm.at[pl.ds(base, n)], ijv)
            pltpu.async_copy(pi_hbm.at[iiv], riv, sem).wait()
            pltpu.async_copy(pj_hbm.at[ijv], riv, sem, add=True).wait()
            pltpu.sync_copy(riv, g_hbm.at[pl.ds(base, n)])

        def body(k, carry):
            chunk(pl.multiple_of(e0 + k * CH, 16), ii, ij, ri, CH)
            return carry

        lax.fori_loop(0, NFULL, body, 0)
        chunk(pl.multiple_of(e0 + NFULL * CH, 16), tii, tij, tri, TAIL)

    @functools.partial(
        pl.kernel,
        out_type=jax.ShapeDtypeStruct((2 * NP, D), f32),
        mesh=mesh,
        scratch_types=[
            pltpu.VMEM((CH,), jnp.int32), pltpu.VMEM((CH, D), f32),
            pltpu.VMEM((TAIL,), jnp.int32), pltpu.VMEM((TAIL, D), f32),
            pltpu.VMEM((ZR, D), f32),
            pltpu.VMEM_SHARED((NP, D), f32),
            pltpu.SemaphoreType.DMA,
        ],
    )
    def _scatter_sc(m_hbm, dst_hbm, zeros_hbm, out_hbm,
                    idx, rows, tidx, trows, stage, acc, sem):
        cid = lax.axis_index("c")
        sid = lax.axis_index("s")
        wid = sid * NC + cid
        r0 = sid * RPW

        pltpu.sync_copy(zeros_hbm, stage)
        for t in range(RPW // ZR):
            pltpu.sync_copy(stage, acc.at[pl.ds(r0 + t * ZR, ZR)])
        plsc.subcore_barrier()

        e0 = wid * EPW

        def chunk(base, iv, rv, n):
            pltpu.sync_copy(dst_hbm.at[pl.ds(base, n)], iv)
            pltpu.sync_copy(m_hbm.at[pl.ds(base, n)], rv)
            pltpu.sync_copy(rv, acc.at[iv], add=True)

        def body(k, carry):
            chunk(pl.multiple_of(e0 + k * CH, 16), idx, rows, CH)
            return carry

        lax.fori_loop(0, NFULL, body, 0)
        chunk(pl.multiple_of(e0 + NFULL * CH, 16), tidx, trows, TAIL)
        plsc.subcore_barrier()

        for t in range(RPW // ZR):
            pltpu.sync_copy(acc.at[pl.ds(r0 + t * ZR, ZR)], stage)
            pltpu.sync_copy(
                stage,
                out_hbm.at[pl.ds(pl.multiple_of(cid * NP + r0 + t * ZR, 8),
                                 ZR)])

    return _gather_sc, _scatter_sc


def _dot(a, b):
    return jnp.dot(a, b, preferred_element_type=f32)


def _ln(h, g, b):
    mu = jnp.mean(h, axis=1, keepdims=True)
    d = h - mu
    var = jnp.mean(d * d, axis=1, keepdims=True)
    return d * lax.rsqrt(var + 1e-5) * g + b


def _mlp3(h, W1, b1, W2, b2, W3, b3):
    h = jax.nn.relu(_dot(h, W1[...]) + b1[...])
    h = jax.nn.relu(_dot(h, W2[...]) + b2[...])
    return _dot(h, W3[...]) + b3[...]


def _enc_node_body(x, W1, b1, W2, b2, W3, b3, g, be, Wi, Wj, xo, pio, pjo):
    xn = _ln(_mlp3(x[...], W1, b1, W2, b2, W3, b3), g[...], be[...])
    xo[...] = xn
    pio[...] = _dot(xn, Wi[...])
    pjo[...] = _dot(xn, Wj[...])


def _edge1_body(ea, gv,
                eW1, eb1, eW2, eb2, eW3, eb3, eg, ebe,
                W1e, b1, W2, b2, W3, b3, g, be, mo):
    eh = _ln(_mlp3(ea[...], eW1, eb1, eW2, eb2, eW3, eb3), eg[...], ebe[...])
    h = jax.nn.relu(_dot(eh, W1e[...]) + gv[...] + b1[...])
    h = jax.nn.relu(_dot(h, W2[...]) + b2[...])
    h = _dot(h, W3[...]) + b3[...]
    mo[...] = _ln(h, g[...], be[...])


def _edge_body(eh, gv, W1e, b1, W2, b2, W3, b3, g, be, mo):
    h = jax.nn.relu(_dot(eh[...], W1e[...]) + gv[...] + b1[...])
    h = jax.nn.relu(_dot(h, W2[...]) + b2[...])
    h = _dot(h, W3[...]) + b3[...]
    mo[...] = _ln(h, g[...], be[...])


def _node_upd_body(p0, p1, x, W1a, W1x, b1, W2, b2, W3, b3, g, be, Wi, Wj,
                   xo, pio, pjo):
    agg = p0[...] + p1[...]
    h = jax.nn.relu(_dot(agg, W1a[...]) + _dot(x[...], W1x[...]) + b1[...])
    h = jax.nn.relu(_dot(h, W2[...]) + b2[...])
    h = _dot(h, W3[...]) + b3[...]
    xn = _ln(h, g[...], be[...])
    xo[...] = xn
    pio[...] = _dot(xn, Wi[...])
    pjo[...] = _dot(xn, Wj[...])


def _node_dec_body(p0, p1, x, W1a, W1x, b1, W2, b2, W3, b3, g, be,
                   dW1, db1, dW2, db2, dW3, db3, oo):
    agg = p0[...] + p1[...]
    h = jax.nn.relu(_dot(agg, W1a[...]) + _dot(x[...], W1x[...]) + b1[...])
    h = jax.nn.relu(_dot(h, W2[...]) + b2[...])
    h = _dot(h, W3[...]) + b3[...]
    xn = _ln(h, g[...], be[...])
    oo[...] = _mlp3(xn, dW1, db1, dW2, db2, dW3, db3)


def _rows(blk, off_blocks=0):
    if off_blocks:
        return pl.BlockSpec((blk, D), lambda i: (off_blocks + i, 0))
    return pl.BlockSpec((blk, D), lambda i: (i, 0))


def _wspec(arr):
    nd = arr.ndim
    return pl.BlockSpec(arr.shape, lambda i: (0,) * nd)


def _tc_call(body, grid, row_ins, weights, n_out, out_rows, blk):
    return pl.pallas_call(
        body,
        grid=(grid,),
        in_specs=list(row_ins) + [_wspec(w) for w in weights],
        out_specs=[_rows(blk)] * n_out,
        out_shape=[jax.ShapeDtypeStruct((out_rows, D), f32)] * n_out,
    )


def kernel(x, edge_index, edge_attr, params):
    src = edge_index[0]
    dst = edge_index[1]

    def vw(v):
        return v.reshape(1, D)

    def unpack(mp, ln=True):
        (W1, b1), (W2, b2), (W3, b3) = mp["layers"]
        out = [W1, vw(b1), W2, vw(b2), W3, vw(b3)]
        if ln:
            g, be = mp["ln"]
            out += [vw(g), vw(be)]
        return out

    enc_n = unpack(params["node_enc"])
    enc_e = unpack(params["edge_enc"])
    upd_e = unpack(params["edge_upd"])
    upd_n = unpack(params["node_upd"])
    dec = unpack(params["node_dec"], ln=False)

    W1u = upd_e[0]
    W1e, W1i, W1j = W1u[:D], W1u[D:2 * D], W1u[2 * D:]
    upd_e_k = [W1e] + upd_e[1:]
    W1n = upd_n[0]
    upd_n_k = [W1n[:D], W1n[D:]] + upd_n[1:]
    zeros = jnp.zeros((ZR, D), f32)

    ng = N // NBLK
    eg = E // EBLK

    enc_node = _tc_call(_enc_node_body, ng, [_rows(NBLK)],
                        enc_n + [W1i, W1j], 3, N, NBLK)
    ea_spec = pl.BlockSpec((EBLK, edge_attr.shape[1]), lambda i: (i, 0))
    edge1 = _tc_call(_edge1_body, eg, [ea_spec, _rows(EBLK)],
                     enc_e + upd_e_k, 1, E, EBLK)
    edge = _tc_call(_edge_body, eg, [_rows(EBLK)] * 2, upd_e_k, 1, E, EBLK)
    node_upd = _tc_call(_node_upd_body, ng,
                        [_rows(NBLK), _rows(NBLK), _rows(NBLK)],
                        upd_n_k + [W1i, W1j], 3, N, NBLK)
    node_dec = _tc_call(_node_dec_body, ng,
                        [_rows(NBLK), _rows(NBLK), _rows(NBLK)],
                        upd_n_k + dec, 1, N, NBLK)

    gather_sc, scatter_sc = _sc_kernels()

    X, Pi, Pj = enc_node(x, *enc_n, W1i, W1j)
    m = None
    out = None
    for step in range(4):
        G = gather_sc(Pi, Pj, dst, src)
        if step == 0:
            (m,) = edge1(edge_attr, G, *enc_e, *upd_e_k)
        else:
            (m,) = edge(m, G, *upd_e_k)
        parts = scatter_sc(m, dst, zeros)
        p0 = lax.slice(parts, (0, 0), (N, D))
        p1 = lax.slice(parts, (NP, 0), (NP + N, D))
        if step < 3:
            X, Pi, Pj = node_upd(p0, p1, X, *upd_n_k, W1i, W1j)
        else:
            (out,) = node_dec(p0, p1, X, *upd_n_k, *dec)
    return out


# speedup vs baseline: 4.8595x; 1.3177x over previous
import functools

import jax
import jax.numpy as jnp
from jax import lax
from jax.experimental import pallas as pl
from jax.experimental.pallas import tpu as pltpu
from jax.experimental.pallas import tpu_sc as plsc

N = 10000
E = 320000
D = 128
NC, NS = 2, 16
NW = NC * NS
EPW = E // NW
CH = 128
NFULL = EPW // CH
TAIL = EPW - NFULL * CH
NP = 10240
RPW = NP // NS
ZR = 128
NBLK = 2000
EBLK = 4000

f32 = jnp.float32


@functools.cache
def _sc_kernels():
    mesh = plsc.VectorSubcoreMesh(core_axis_name="c", subcore_axis_name="s",
                                  num_cores=NC, num_subcores=NS)

    @functools.partial(
        pl.kernel,
        out_type=jax.ShapeDtypeStruct((E, D), f32),
        mesh=mesh,
        scratch_types=[
            pltpu.VMEM((CH,), jnp.int32), pltpu.VMEM((CH,), jnp.int32),
            pltpu.VMEM((CH,), jnp.int32), pltpu.VMEM((CH,), jnp.int32),
            pltpu.VMEM((CH, D), f32), pltpu.VMEM((CH, D), f32),
            pltpu.VMEM((TAIL,), jnp.int32), pltpu.VMEM((TAIL,), jnp.int32),
            pltpu.VMEM((TAIL, D), f32),
            pltpu.SemaphoreType.DMA, pltpu.SemaphoreType.DMA,
            pltpu.SemaphoreType.DMA, pltpu.SemaphoreType.DMA,
            pltpu.SemaphoreType.DMA, pltpu.SemaphoreType.DMA,
        ],
    )
    def _gather_sc(pi_hbm, pj_hbm, dst_hbm, src_hbm, g_hbm,
                   ii0, ij0, ii1, ij1, ri0, ri1, tii, tij, tri,
                   sI0, sI1, sG0, sG1, sO0, sO1):
        wid = lax.axis_index("s") * NC + lax.axis_index("c")
        e0 = wid * EPW
        last0 = e0 + (NFULL - 2) * CH
        last1 = e0 + (NFULL - 1) * CH

        def idx_load(base, iiv, ijv, sem):
            pltpu.async_copy(dst_hbm.at[pl.ds(base, CH)], iiv, sem)
            pltpu.async_copy(src_hbm.at[pl.ds(base, CH)], ijv, sem)

        def wait_idx(iiv, ijv, sem):
            pltpu.make_async_copy(dst_hbm.at[pl.ds(0, CH)], iiv, sem).wait()
            pltpu.make_async_copy(src_hbm.at[pl.ds(0, CH)], ijv, sem).wait()

        def wait_rows(riv, sem):
            pltpu.make_async_copy(pi_hbm.at[pl.ds(0, CH)], riv, sem).wait()

        def wait_out(riv, sem):
            pltpu.make_async_copy(riv, g_hbm.at[pl.ds(0, CH)], sem).wait()

        idx_load(pl.multiple_of(e0, 16), ii0, ij0, sI0)
        idx_load(pl.multiple_of(e0 + CH, 16), ii1, ij1, sI1)

        def pair(k, carry):
            c0 = pl.multiple_of(e0 + (2 * k) * CH, 16)
            c1 = pl.multiple_of(e0 + (2 * k + 1) * CH, 16)
            wait_idx(ii0, ij0, sI0)

            @pl.when(k > 0)
            def _():
                wait_out(ri0, sO0)

            pltpu.async_copy(pi_hbm.at[ii0], ri0, sG0)
            wait_idx(ii1, ij1, sI1)

            @pl.when(k > 0)
            def _():
                wait_out(ri1, sO1)

            wait_rows(ri0, sG0)
            pltpu.async_copy(pj_hbm.at[ij0], ri0, sG0, add=True)
            pltpu.async_copy(pi_hbm.at[ii1], ri1, sG1)
            wait_rows(ri0, sG0)
            pltpu.async_copy(ri0, g_hbm.at[pl.ds(c0, CH)], sO0)
            nb0 = pl.multiple_of(
                jnp.minimum(e0 + (2 * k + 2) * CH, last0), 16)
            idx_load(nb0, ii0, ij0, sI0)
            wait_rows(ri1, sG1)
            pltpu.async_copy(pj_hbm.at[ij1], ri1, sG1, add=True)
            wait_rows(ri1, sG1)
            pltpu.async_copy(ri1, g_hbm.at[pl.ds(c1, CH)], sO1)
            nb1 = pl.multiple_of(
                jnp.minimum(e0 + (2 * k + 3) * CH, last1), 16)
            idx_load(nb1, ii1, ij1, sI1)
            return carry

        lax.fori_loop(0, NFULL // 2, pair, 0)
        wait_idx(ii0, ij0, sI0)
        wait_idx(ii1, ij1, sI1)
        wait_out(ri0, sO0)
        wait_out(ri1, sO1)

        tb = pl.multiple_of(e0 + NFULL * CH, 16)
        pltpu.sync_copy(dst_hbm.at[pl.ds(tb, TAIL)], tii)
        pltpu.sync_copy(src_hbm.at[pl.ds(tb, TAIL)], tij)
        pltpu.async_copy(pi_hbm.at[tii], tri, sG0).wait()
        pltpu.async_copy(pj_hbm.at[tij], tri, sG0, add=True).wait()
        pltpu.sync_copy(tri, g_hbm.at[pl.ds(tb, TAIL)])

    @functools.partial(
        pl.kernel,
        out_type=jax.ShapeDtypeStruct((2 * NP, D), f32),
        mesh=mesh,
        scratch_types=[
            pltpu.VMEM((CH,), jnp.int32), pltpu.VMEM((CH, D), f32),
            pltpu.VMEM((CH,), jnp.int32), pltpu.VMEM((CH, D), f32),
            pltpu.VMEM((TAIL,), jnp.int32), pltpu.VMEM((TAIL, D), f32),
            pltpu.VMEM_SHARED((NP, D), f32),
            pltpu.SemaphoreType.DMA, pltpu.SemaphoreType.DMA,
            pltpu.SemaphoreType.DMA, pltpu.SemaphoreType.DMA,
        ],
    )
    def _scatter_sc(m_hbm, dst_hbm, zeros_hbm, out_hbm,
                    i0, r0_, i1, r1_, tidx, trows, acc,
                    sA0, sA1, sF0, sF1):
        cid = lax.axis_index("c")
        sid = lax.axis_index("s")
        wid = sid * NC + cid
        r0 = sid * RPW

        pltpu.sync_copy(zeros_hbm, r0_)
        for t in range(RPW // ZR):
            pltpu.sync_copy(r0_, acc.at[pl.ds(r0 + t * ZR, ZR)])
        plsc.subcore_barrier()

        e0 = wid * EPW
        last0 = e0 + (NFULL - 2) * CH
        last1 = e0 + (NFULL - 1) * CH

        def load_pair(base, iv, rv, sem):
            pltpu.async_copy(dst_hbm.at[pl.ds(base, CH)], iv, sem)
            pltpu.async_copy(m_hbm.at[pl.ds(base, CH)], rv, sem)

        def wait_load(iv, rv, sem):
            pltpu.make_async_copy(dst_hbm.at[pl.ds(0, CH)], iv, sem).wait()
            pltpu.make_async_copy(m_hbm.at[pl.ds(0, CH)], rv, sem).wait()

        def wait_scat(iv, rv, sem):
            pltpu.make_async_copy(rv, acc.at[iv], sem).wait()

        load_pair(pl.multiple_of(e0, 16), i0, r0_, sA0)
        load_pair(pl.multiple_of(e0 + CH, 16), i1, r1_, sA1)

        def pair(k, carry):
            wait_load(i0, r0_, sA0)
            pltpu.async_copy(r0_, acc.at[i0], sF0, add=True)
            wait_load(i1, r1_, sA1)
            pltpu.async_copy(r1_, acc.at[i1], sF1, add=True)
            wait_scat(i0, r0_, sF0)
            nb0 = pl.multiple_of(
                jnp.minimum(e0 + (2 * k + 2) * CH, last0), 16)
            load_pair(nb0, i0, r0_, sA0)
            wait_scat(i1, r1_, sF1)
            nb1 = pl.multiple_of(
                jnp.minimum(e0 + (2 * k + 3) * CH, last1), 16)
            load_pair(nb1, i1, r1_, sA1)
            return carry

        lax.fori_loop(0, NFULL // 2, pair, 0)
        wait_load(i0, r0_, sA0)
        wait_load(i1, r1_, sA1)

        tb = pl.multiple_of(e0 + NFULL * CH, 16)
        pltpu.sync_copy(dst_hbm.at[pl.ds(tb, TAIL)], tidx)
        pltpu.sync_copy(m_hbm.at[pl.ds(tb, TAIL)], trows)
        pltpu.sync_copy(trows, acc.at[tidx], add=True)
        plsc.subcore_barrier()

        for t in range(RPW // ZR):
            pltpu.sync_copy(acc.at[pl.ds(r0 + t * ZR, ZR)], r0_)
            pltpu.sync_copy(
                r0_,
                out_hbm.at[pl.ds(pl.multiple_of(cid * NP + r0 + t * ZR, 8),
                                 ZR)])

    return _gather_sc, _scatter_sc


def _dot(a, b):
    return jnp.dot(a, b, preferred_element_type=f32)


def _ln(h, g, b):
    mu = jnp.mean(h, axis=1, keepdims=True)
    d = h - mu
    var = jnp.mean(d * d, axis=1, keepdims=True)
    return d * lax.rsqrt(var + 1e-5) * g + b


def _mlp3(h, W1, b1, W2, b2, W3, b3):
    h = jax.nn.relu(_dot(h, W1[...]) + b1[...])
    h = jax.nn.relu(_dot(h, W2[...]) + b2[...])
    return _dot(h, W3[...]) + b3[...]


def _enc_node_body(x, W1, b1, W2, b2, W3, b3, g, be, Wi, Wj, xo, pio, pjo):
    xn = _ln(_mlp3(x[...], W1, b1, W2, b2, W3, b3), g[...], be[...])
    xo[...] = xn
    pio[...] = _dot(xn, Wi[...])
    pjo[...] = _dot(xn, Wj[...])


def _edge1_body(ea, gv,
                eW1, eb1, eW2, eb2, eW3, eb3, eg, ebe,
                W1e, b1, W2, b2, W3, b3, g, be, mo):
    eh = _ln(_mlp3(ea[...], eW1, eb1, eW2, eb2, eW3, eb3), eg[...], ebe[...])
    h = jax.nn.relu(_dot(eh, W1e[...]) + gv[...] + b1[...])
    h = jax.nn.relu(_dot(h, W2[...]) + b2[...])
    h = _dot(h, W3[...]) + b3[...]
    mo[...] = _ln(h, g[...], be[...])


def _edge_body(eh, gv, W1e, b1, W2, b2, W3, b3, g, be, mo):
    h = jax.nn.relu(_dot(eh[...], W1e[...]) + gv[...] + b1[...])
    h = jax.nn.relu(_dot(h, W2[...]) + b2[...])
    h = _dot(h, W3[...]) + b3[...]
    mo[...] = _ln(h, g[...], be[...])


def _node_upd_body(p0, p1, x, W1a, W1x, b1, W2, b2, W3, b3, g, be, Wi, Wj,
                   xo, pio, pjo):
    agg = p0[...] + p1[...]
    h = jax.nn.relu(_dot(agg, W1a[...]) + _dot(x[...], W1x[...]) + b1[...])
    h = jax.nn.relu(_dot(h, W2[...]) + b2[...])
    h = _dot(h, W3[...]) + b3[...]
    xn = _ln(h, g[...], be[...])
    xo[...] = xn
    pio[...] = _dot(xn, Wi[...])
    pjo[...] = _dot(xn, Wj[...])


def _node_dec_body(p0, p1, x, W1a, W1x, b1, W2, b2, W3, b3, g, be,
                   dW1, db1, dW2, db2, dW3, db3, oo):
    agg = p0[...] + p1[...]
    h = jax.nn.relu(_dot(agg, W1a[...]) + _dot(x[...], W1x[...]) + b1[...])
    h = jax.nn.relu(_dot(h, W2[...]) + b2[...])
    h = _dot(h, W3[...]) + b3[...]
    xn = _ln(h, g[...], be[...])
    oo[...] = _mlp3(xn, dW1, db1, dW2, db2, dW3, db3)


def _rows(blk, off_blocks=0):
    if off_blocks:
        return pl.BlockSpec((blk, D), lambda i: (off_blocks + i, 0))
    return pl.BlockSpec((blk, D), lambda i: (i, 0))


def _wspec(arr):
    nd = arr.ndim
    return pl.BlockSpec(arr.shape, lambda i: (0,) * nd)


def _tc_call(body, grid, row_ins, weights, n_out, out_rows, blk):
    return pl.pallas_call(
        body,
        grid=(grid,),
        in_specs=list(row_ins) + [_wspec(w) for w in weights],
        out_specs=[_rows(blk)] * n_out,
        out_shape=[jax.ShapeDtypeStruct((out_rows, D), f32)] * n_out,
    )


def kernel(x, edge_index, edge_attr, params):
    src = edge_index[0]
    dst = edge_index[1]

    def vw(v):
        return v.reshape(1, D)

    def unpack(mp, ln=True):
        (W1, b1), (W2, b2), (W3, b3) = mp["layers"]
        out = [W1, vw(b1), W2, vw(b2), W3, vw(b3)]
        if ln:
            g, be = mp["ln"]
            out += [vw(g), vw(be)]
        return out

    enc_n = unpack(params["node_enc"])
    enc_e = unpack(params["edge_enc"])
    upd_e = unpack(params["edge_upd"])
    upd_n = unpack(params["node_upd"])
    dec = unpack(params["node_dec"], ln=False)

    W1u = upd_e[0]
    W1e, W1i, W1j = W1u[:D], W1u[D:2 * D], W1u[2 * D:]
    upd_e_k = [W1e] + upd_e[1:]
    W1n = upd_n[0]
    upd_n_k = [W1n[:D], W1n[D:]] + upd_n[1:]
    zeros = jnp.zeros((ZR, D), f32)

    ng = N // NBLK
    eg = E // EBLK

    enc_node = _tc_call(_enc_node_body, ng, [_rows(NBLK)],
                        enc_n + [W1i, W1j], 3, N, NBLK)
    ea_spec = pl.BlockSpec((EBLK, edge_attr.shape[1]), lambda i: (i, 0))
    edge1 = _tc_call(_edge1_body, eg, [ea_spec, _rows(EBLK)],
                     enc_e + upd_e_k, 1, E, EBLK)
    edge = _tc_call(_edge_body, eg, [_rows(EBLK)] * 2, upd_e_k, 1, E, EBLK)
    node_upd = _tc_call(_node_upd_body, ng,
                        [_rows(NBLK), _rows(NBLK), _rows(NBLK)],
                        upd_n_k + [W1i, W1j], 3, N, NBLK)
    node_dec = _tc_call(_node_dec_body, ng,
                        [_rows(NBLK), _rows(NBLK), _rows(NBLK)],
                        upd_n_k + dec, 1, N, NBLK)

    gather_sc, scatter_sc = _sc_kernels()

    X, Pi, Pj = enc_node(x, *enc_n, W1i, W1j)
    m = None
    out = None
    for step in range(4):
        G = gather_sc(Pi, Pj, dst, src)
        if step == 0:
            (m,) = edge1(edge_attr, G, *enc_e, *upd_e_k)
        else:
            (m,) = edge(m, G, *upd_e_k)
        parts = scatter_sc(m, dst, zeros)
        p0 = lax.slice(parts, (0, 0), (N, D))
        p1 = lax.slice(parts, (NP, 0), (NP + N, D))
        if step < 3:
            X, Pi, Pj = node_upd(p0, p1, X, *upd_n_k, W1i, W1j)
        else:
            (out,) = node_dec(p0, p1, X, *upd_n_k, *dec)
    return out


# speedup vs baseline: 5.4183x; 1.1150x over previous
import functools

import jax
import jax.numpy as jnp
from jax import lax
from jax.experimental import pallas as pl
from jax.experimental.pallas import tpu as pltpu
from jax.experimental.pallas import tpu_sc as plsc

N = 10000
E = 320000
D = 128
NC, NS = 2, 16
NW = NC * NS
EH = E // 2
CH = 128
NP = 10240
RPW = NP // NS
ZR = 128
NBLK = 2000
EBLK = 4000

f32 = jnp.float32


@functools.cache
def _sc_kernels(ne):
    mesh = plsc.VectorSubcoreMesh(core_axis_name="c", subcore_axis_name="s",
                                  num_cores=NC, num_subcores=NS)
    EPW = ne // NW
    NFULL = EPW // CH
    TAIL = EPW - NFULL * CH
    AL = 8

    @functools.partial(
        pl.kernel,
        out_type=jax.ShapeDtypeStruct((ne, D), f32),
        mesh=mesh,
        scratch_types=[
            pltpu.VMEM((CH,), jnp.int32), pltpu.VMEM((CH,), jnp.int32),
            pltpu.VMEM((CH,), jnp.int32), pltpu.VMEM((CH,), jnp.int32),
            pltpu.VMEM((CH, D), f32), pltpu.VMEM((CH, D), f32),
            pltpu.VMEM((max(TAIL, 8),), jnp.int32),
            pltpu.VMEM((max(TAIL, 8),), jnp.int32),
            pltpu.VMEM((max(TAIL, 8), D), f32),
            pltpu.SemaphoreType.DMA, pltpu.SemaphoreType.DMA,
            pltpu.SemaphoreType.DMA, pltpu.SemaphoreType.DMA,
            pltpu.SemaphoreType.DMA, pltpu.SemaphoreType.DMA,
        ],
    )
    def _gather_sc(pi_hbm, pj_hbm, dst_hbm, src_hbm, g_hbm,
                   ii0, ij0, ii1, ij1, ri0, ri1, tii, tij, tri,
                   sI0, sI1, sG0, sG1, sO0, sO1):
        wid = lax.axis_index("s") * NC + lax.axis_index("c")
        e0 = wid * EPW
        last0 = e0 + (NFULL - 2) * CH
        last1 = e0 + (NFULL - 1) * CH

        def idx_load(base, iiv, ijv, sem):
            pltpu.async_copy(dst_hbm.at[pl.ds(base, CH)], iiv, sem)
            pltpu.async_copy(src_hbm.at[pl.ds(base, CH)], ijv, sem)

        def wait_idx(iiv, ijv, sem):
            pltpu.make_async_copy(dst_hbm.at[pl.ds(0, CH)], iiv, sem).wait()
            pltpu.make_async_copy(src_hbm.at[pl.ds(0, CH)], ijv, sem).wait()

        def wait_rows(riv, sem):
            pltpu.make_async_copy(pi_hbm.at[pl.ds(0, CH)], riv, sem).wait()

        def wait_out(riv, sem):
            pltpu.make_async_copy(riv, g_hbm.at[pl.ds(0, CH)], sem).wait()

        idx_load(pl.multiple_of(e0, AL), ii0, ij0, sI0)
        idx_load(pl.multiple_of(e0 + CH, AL), ii1, ij1, sI1)

        def pair(k, carry):
            c0 = pl.multiple_of(e0 + (2 * k) * CH, AL)
            c1 = pl.multiple_of(e0 + (2 * k + 1) * CH, AL)
            wait_idx(ii0, ij0, sI0)

            @pl.when(k > 0)
            def _():
                wait_out(ri0, sO0)

            pltpu.async_copy(pi_hbm.at[ii0], ri0, sG0)
            wait_idx(ii1, ij1, sI1)

            @pl.when(k > 0)
            def _():
                wait_out(ri1, sO1)

            wait_rows(ri0, sG0)
            pltpu.async_copy(pj_hbm.at[ij0], ri0, sG0, add=True)
            pltpu.async_copy(pi_hbm.at[ii1], ri1, sG1)
            wait_rows(ri0, sG0)
            pltpu.async_copy(ri0, g_hbm.at[pl.ds(c0, CH)], sO0)
            nb0 = pl.multiple_of(
                jnp.minimum(e0 + (2 * k + 2) * CH, last0), AL)
            idx_load(nb0, ii0, ij0, sI0)
            wait_rows(ri1, sG1)
            pltpu.async_copy(pj_hbm.at[ij1], ri1, sG1, add=True)
            wait_rows(ri1, sG1)
            pltpu.async_copy(ri1, g_hbm.at[pl.ds(c1, CH)], sO1)
            nb1 = pl.multiple_of(
                jnp.minimum(e0 + (2 * k + 3) * CH, last1), AL)
            idx_load(nb1, ii1, ij1, sI1)
            return carry

        lax.fori_loop(0, NFULL // 2, pair, 0)
        wait_idx(ii0, ij0, sI0)
        wait_idx(ii1, ij1, sI1)
        wait_out(ri0, sO0)
        wait_out(ri1, sO1)

        if NFULL % 2:
            ob = pl.multiple_of(e0 + (NFULL - 1) * CH, AL)
            pltpu.sync_copy(dst_hbm.at[pl.ds(ob, CH)], ii0)
            pltpu.sync_copy(src_hbm.at[pl.ds(ob, CH)], ij0)
            pltpu.async_copy(pi_hbm.at[ii0], ri0, sG0).wait()
            pltpu.async_copy(pj_hbm.at[ij0], ri0, sG0, add=True).wait()
            pltpu.sync_copy(ri0, g_hbm.at[pl.ds(ob, CH)])

        if TAIL:
            tb = pl.multiple_of(e0 + NFULL * CH, AL)
            pltpu.sync_copy(dst_hbm.at[pl.ds(tb, TAIL)], tii)
            pltpu.sync_copy(src_hbm.at[pl.ds(tb, TAIL)], tij)
            pltpu.async_copy(pi_hbm.at[tii], tri, sG0).wait()
            pltpu.async_copy(pj_hbm.at[tij], tri, sG0, add=True).wait()
            pltpu.sync_copy(tri, g_hbm.at[pl.ds(tb, TAIL)])

    @functools.partial(
        pl.kernel,
        out_type=jax.ShapeDtypeStruct((2 * NP, D), f32),
        mesh=mesh,
        scratch_types=[
            pltpu.VMEM((CH,), jnp.int32), pltpu.VMEM((CH, D), f32),
            pltpu.VMEM((CH,), jnp.int32), pltpu.VMEM((CH, D), f32),
            pltpu.VMEM((max(TAIL, 8),), jnp.int32),
            pltpu.VMEM((max(TAIL, 8), D), f32),
            pltpu.VMEM_SHARED((NP, D), f32),
            pltpu.SemaphoreType.DMA, pltpu.SemaphoreType.DMA,
            pltpu.SemaphoreType.DMA, pltpu.SemaphoreType.DMA,
        ],
    )
    def _scatter_sc(m_hbm, dst_hbm, zeros_hbm, out_hbm,
                    i0, r0_, i1, r1_, tidx, trows, acc,
                    sA0, sA1, sF0, sF1):
        cid = lax.axis_index("c")
        sid = lax.axis_index("s")
        wid = sid * NC + cid
        r0 = sid * RPW

        pltpu.sync_copy(zeros_hbm, r0_)
        for t in range(RPW // ZR):
            pltpu.sync_copy(r0_, acc.at[pl.ds(r0 + t * ZR, ZR)])
        plsc.subcore_barrier()

        e0 = wid * EPW
        last0 = e0 + (NFULL - 2) * CH
        last1 = e0 + (NFULL - 1) * CH

        def load_pair(base, iv, rv, sem):
            pltpu.async_copy(dst_hbm.at[pl.ds(base, CH)], iv, sem)
            pltpu.async_copy(m_hbm.at[pl.ds(base, CH)], rv, sem)

        def wait_load(iv, rv, sem):
            pltpu.make_async_copy(dst_hbm.at[pl.ds(0, CH)], iv, sem).wait()
            pltpu.make_async_copy(m_hbm.at[pl.ds(0, CH)], rv, sem).wait()

        def wait_scat(iv, rv, sem):
            pltpu.make_async_copy(rv, acc.at[iv], sem).wait()

        load_pair(pl.multiple_of(e0, AL), i0, r0_, sA0)
        load_pair(pl.multiple_of(e0 + CH, AL), i1, r1_, sA1)

        def pair(k, carry):
            wait_load(i0, r0_, sA0)
            pltpu.async_copy(r0_, acc.at[i0], sF0, add=True)
            wait_load(i1, r1_, sA1)
            pltpu.async_copy(r1_, acc.at[i1], sF1, add=True)
            wait_scat(i0, r0_, sF0)
            nb0 = pl.multiple_of(
                jnp.minimum(e0 + (2 * k + 2) * CH, last0), AL)
            load_pair(nb0, i0, r0_, sA0)
            wait_scat(i1, r1_, sF1)
            nb1 = pl.multiple_of(
                jnp.minimum(e0 + (2 * k + 3) * CH, last1), AL)
            load_pair(nb1, i1, r1_, sA1)
            return carry

        lax.fori_loop(0, NFULL // 2, pair, 0)
        wait_load(i0, r0_, sA0)
        wait_load(i1, r1_, sA1)

        if NFULL % 2:
            ob = pl.multiple_of(e0 + (NFULL - 1) * CH, AL)
            pltpu.sync_copy(dst_hbm.at[pl.ds(ob, CH)], i0)
            pltpu.sync_copy(m_hbm.at[pl.ds(ob, CH)], r0_)
            pltpu.sync_copy(r0_, acc.at[i0], add=True)

        if TAIL:
            tb = pl.multiple_of(e0 + NFULL * CH, AL)
            pltpu.sync_copy(dst_hbm.at[pl.ds(tb, TAIL)], tidx)
            pltpu.sync_copy(m_hbm.at[pl.ds(tb, TAIL)], trows)
            pltpu.sync_copy(trows, acc.at[tidx], add=True)
        plsc.subcore_barrier()

        for t in range(RPW // ZR):
            pltpu.sync_copy(acc.at[pl.ds(r0 + t * ZR, ZR)], r0_)
            pltpu.sync_copy(
                r0_,
                out_hbm.at[pl.ds(pl.multiple_of(cid * NP + r0 + t * ZR, 8),
                                 ZR)])

    return _gather_sc, _scatter_sc


def _dot(a, b):
    return jnp.dot(a, b, preferred_element_type=f32)


def _ln(h, g, b):
    mu = jnp.mean(h, axis=1, keepdims=True)
    d = h - mu
    var = jnp.mean(d * d, axis=1, keepdims=True)
    return d * lax.rsqrt(var + 1e-5) * g + b


def _mlp3(h, W1, b1, W2, b2, W3, b3):
    h = jax.nn.relu(_dot(h, W1[...]) + b1[...])
    h = jax.nn.relu(_dot(h, W2[...]) + b2[...])
    return _dot(h, W3[...]) + b3[...]


def _enc_node_body(x, W1, b1, W2, b2, W3, b3, g, be, Wi, Wj, xo, pio, pjo):
    xn = _ln(_mlp3(x[...], W1, b1, W2, b2, W3, b3), g[...], be[...])
    xo[...] = xn
    pio[...] = _dot(xn, Wi[...])
    pjo[...] = _dot(xn, Wj[...])


def _edge1_body(ea, gv,
                eW1, eb1, eW2, eb2, eW3, eb3, eg, ebe,
                W1e, b1, W2, b2, W3, b3, g, be, mo):
    eh = _ln(_mlp3(ea[...], eW1, eb1, eW2, eb2, eW3, eb3), eg[...], ebe[...])
    h = jax.nn.relu(_dot(eh, W1e[...]) + gv[...] + b1[...])
    h = jax.nn.relu(_dot(h, W2[...]) + b2[...])
    h = _dot(h, W3[...]) + b3[...]
    mo[...] = _ln(h, g[...], be[...])


def _edge_body(eh, gv, W1e, b1, W2, b2, W3, b3, g, be, mo):
    h = jax.nn.relu(_dot(eh[...], W1e[...]) + gv[...] + b1[...])
    h = jax.nn.relu(_dot(h, W2[...]) + b2[...])
    h = _dot(h, W3[...]) + b3[...]
    mo[...] = _ln(h, g[...], be[...])


def _node_upd_body(p0, p1, p2, p3, x, W1a, W1x, b1, W2, b2, W3, b3, g, be,
                   Wi, Wj, xo, pio, pjo):
    agg = (p0[...] + p1[...]) + (p2[...] + p3[...])
    h = jax.nn.relu(_dot(agg, W1a[...]) + _dot(x[...], W1x[...]) + b1[...])
    h = jax.nn.relu(_dot(h, W2[...]) + b2[...])
    h = _dot(h, W3[...]) + b3[...]
    xn = _ln(h, g[...], be[...])
    xo[...] = xn
    pio[...] = _dot(xn, Wi[...])
    pjo[...] = _dot(xn, Wj[...])


def _node_dec_body(p0, p1, p2, p3, x, W1a, W1x, b1, W2, b2, W3, b3, g, be,
                   dW1, db1, dW2, db2, dW3, db3, oo):
    agg = (p0[...] + p1[...]) + (p2[...] + p3[...])
    h = jax.nn.relu(_dot(agg, W1a[...]) + _dot(x[...], W1x[...]) + b1[...])
    h = jax.nn.relu(_dot(h, W2[...]) + b2[...])
    h = _dot(h, W3[...]) + b3[...]
    xn = _ln(h, g[...], be[...])
    oo[...] = _mlp3(xn, dW1, db1, dW2, db2, dW3, db3)


def _rows(blk, off_blocks=0):
    if off_blocks:
        return pl.BlockSpec((blk, D), lambda i: (off_blocks + i, 0))
    return pl.BlockSpec((blk, D), lambda i: (i, 0))


def _wspec(arr):
    nd = arr.ndim
    return pl.BlockSpec(arr.shape, lambda i: (0,) * nd)


def _tc_call(body, grid, row_ins, weights, n_out, out_rows, blk):
    return pl.pallas_call(
        body,
        grid=(grid,),
        in_specs=list(row_ins) + [_wspec(w) for w in weights],
        out_specs=[_rows(blk)] * n_out,
        out_shape=[jax.ShapeDtypeStruct((out_rows, D), f32)] * n_out,
    )


def kernel(x, edge_index, edge_attr, params):
    src = edge_index[0]
    dst = edge_index[1]

    def vw(v):
        return v.reshape(1, D)

    def unpack(mp, ln=True):
        (W1, b1), (W2, b2), (W3, b3) = mp["layers"]
        out = [W1, vw(b1), W2, vw(b2), W3, vw(b3)]
        if ln:
            g, be = mp["ln"]
            out += [vw(g), vw(be)]
        return out

    enc_n = unpack(params["node_enc"])
    enc_e = unpack(params["edge_enc"])
    upd_e = unpack(params["edge_upd"])
    upd_n = unpack(params["node_upd"])
    dec = unpack(params["node_dec"], ln=False)

    W1u = upd_e[0]
    W1e, W1i, W1j = W1u[:D], W1u[D:2 * D], W1u[2 * D:]
    upd_e_k = [W1e] + upd_e[1:]
    W1n = upd_n[0]
    upd_n_k = [W1n[:D], W1n[D:]] + upd_n[1:]
    zeros = jnp.zeros((ZR, D), f32)

    ng = N // NBLK
    eg = EH // EBLK

    enc_node = _tc_call(_enc_node_body, ng, [_rows(NBLK)],
                        enc_n + [W1i, W1j], 3, N, NBLK)
    ea_spec = pl.BlockSpec((EBLK, edge_attr.shape[1]), lambda i: (i, 0))
    edge1 = _tc_call(_edge1_body, eg, [ea_spec, _rows(EBLK)],
                     enc_e + upd_e_k, 1, EH, EBLK)
    edge = _tc_call(_edge_body, eg, [_rows(EBLK)] * 2, upd_e_k, 1, EH, EBLK)
    node_upd = _tc_call(_node_upd_body, ng,
                        [_rows(NBLK)] * 5,
                        upd_n_k + [W1i, W1j], 3, N, NBLK)
    node_dec = _tc_call(_node_dec_body, ng,
                        [_rows(NBLK)] * 5,
                        upd_n_k + dec, 1, N, NBLK)

    gather_sc, scatter_sc = _sc_kernels(EH)

    dstA = lax.slice(dst, (0,), (EH,))
    dstB = lax.slice(dst, (EH,), (E,))
    srcA = lax.slice(src, (0,), (EH,))
    srcB = lax.slice(src, (EH,), (E,))
    eaA = lax.slice(edge_attr, (0, 0), (EH, edge_attr.shape[1]))
    eaB = lax.slice(edge_attr, (EH, 0), (E, edge_attr.shape[1]))

    X, Pi, Pj = enc_node(x, *enc_n, W1i, W1j)
    mA = mB = None
    out = None
    for step in range(4):
        GA = gather_sc(Pi, Pj, dstA, srcA)
        GB = gather_sc(Pi, Pj, dstB, srcB)
        if step == 0:
            (mA,) = edge1(eaA, GA, *enc_e, *upd_e_k)
            (mB,) = edge1(eaB, GB, *enc_e, *upd_e_k)
        else:
            (mA,) = edge(mA, GA, *upd_e_k)
            (mB,) = edge(mB, GB, *upd_e_k)
        partsA = scatter_sc(mA, dstA, zeros)
        partsB = scatter_sc(mB, dstB, zeros)
        p0 = lax.slice(partsA, (0, 0), (N, D))
        p1 = lax.slice(partsA, (NP, 0), (NP + N, D))
        p2 = lax.slice(partsB, (0, 0), (N, D))
        p3 = lax.slice(partsB, (NP, 0), (NP + N, D))
        if step < 3:
            X, Pi, Pj = node_upd(p0, p1, p2, p3, X, *upd_n_k, W1i, W1j)
        else:
            (out,) = node_dec(p0, p1, p2, p3, X, *upd_n_k, *dec)
    return out
